# baseline (speedup 1.0000x reference)
import functools

import jax
import jax.numpy as jnp
from jax import lax
from jax.experimental import pallas as pl
from jax.experimental.pallas import tpu as pltpu
from jax.experimental.pallas import tpu_sc as plsc


def _sc_fused_gather(X_domains, aT, table):
    info = plsc.get_sparse_core_info()
    nc, ns = info.num_cores, info.num_subcores
    nw = nc * ns
    arity, B = aT.shape
    D = table.shape[1]
    bpw = B // nw
    mesh = plsc.VectorSubcoreMesh(core_axis_name="c", subcore_axis_name="s",
                                  num_cores=nc)

    @functools.partial(
        pl.kernel,
        out_type=(jax.ShapeDtypeStruct((B, D), jnp.float32),
                  jax.ShapeDtypeStruct((B, D), jnp.float32)),
        mesh=mesh,
        scratch_types=[
            pltpu.VMEM((arity, bpw), jnp.int32),
            pltpu.VMEM((bpw,), jnp.int32),
            pltpu.VMEM((bpw,), jnp.int32),
            pltpu.VMEM((bpw, D), jnp.float32),
            pltpu.VMEM((bpw, D), jnp.float32),
            pltpu.SemaphoreType.DMA,
            pltpu.SemaphoreType.DMA,
        ],
        compiler_params=pltpu.CompilerParams(use_tc_tiling_on_sc=False),
    )
    def gather_kernel(xdom, a_hbm, tab, eh_out, et_out,
                      a2_v, ih_v, it_v, eh_v, et_v, sem_h, sem_t):
        wid = lax.axis_index("s") * nc + lax.axis_index("c")
        base = wid * bpw
        pltpu.sync_copy(a_hbm.at[:, pl.ds(base, bpw)], a2_v)
        ch = pltpu.async_copy(xdom.at[a2_v.at[0]], ih_v, sem_h)
        ct = pltpu.async_copy(xdom.at[a2_v.at[1]], it_v, sem_t)
        ch.wait()
        gh = pltpu.async_copy(tab.at[ih_v], eh_v, sem_h)
        ct.wait()
        gt = pltpu.async_copy(tab.at[it_v], et_v, sem_t)
        gh.wait()
        pltpu.sync_copy(eh_v, eh_out.at[pl.ds(base, bpw)])
        gt.wait()
        pltpu.sync_copy(et_v, et_out.at[pl.ds(base, bpw)])

    return gather_kernel(X_domains, aT, table)


def _mm_body(eh_ref, et_ref, w_ref, b_ref, o_ref):
    eh = eh_ref[...]
    et = et_ref[...]
    D = eh.shape[1]
    hp = jax.lax.Precision.HIGHEST
    acc = jnp.dot(eh, w_ref[0:D, :], precision=hp,
                  preferred_element_type=jnp.float32)
    acc = acc + jnp.dot(et, w_ref[D:2 * D, :], precision=hp,
                        preferred_element_type=jnp.float32)
    acc = acc + jnp.dot(eh * et, w_ref[2 * D:3 * D, :], precision=hp,
                        preferred_element_type=jnp.float32)
    o_ref[...] = jnp.tanh(acc + b_ref[...])


def _tc_embed(eh, et, W, b):
    B, D = eh.shape
    K, A = W.shape
    blk = 2048
    return pl.pallas_call(
        _mm_body,
        grid=(B // blk,),
        in_specs=[
            pl.BlockSpec((blk, D), lambda i: (i, 0)),
            pl.BlockSpec((blk, D), lambda i: (i, 0)),
            pl.BlockSpec((K, A), lambda i: (0, 0)),
            pl.BlockSpec((A,), lambda i: (0,)),
        ],
        out_specs=pl.BlockSpec((blk, A), lambda i: (i, 0)),
        out_shape=jax.ShapeDtypeStruct((B, A), jnp.float32),
    )(eh, et, W, b)


def kernel(X_domains, A_predicates, table, W, b):
    aT = A_predicates.T
    eh, et = _sc_fused_gather(X_domains, aT, table)
    return _tc_embed(eh, et, W, b)
